# Initial kernel scaffold; baseline (speedup 1.0000x reference)
import jax
import jax.numpy as jnp
from jax.experimental import pallas as pl


def kernel(idxs, feats, ifeats, keys, ikeys, values, scale):
    raise NotImplementedError("write your pallas kernel here")


# speedup vs baseline: 1.1642x; 1.1642x over previous
import jax
import jax.numpy as jnp
from jax.experimental import pallas as pl

D0 = 16384
D1 = 16384
S = 32768
F = 64
NQ = 16


def _quant_body(x_ref, k_ref, o_ref):
    tf = jax.nn.sigmoid(x_ref[...])
    k = jax.nn.sigmoid(k_ref[...])
    best = jnp.full_like(tf, jnp.inf)
    val = jnp.zeros_like(tf)
    for c in range(NQ):
        kc = k[c][None, :]
        d = jnp.abs(tf - kc)
        m = d < best
        best = jnp.where(m, d, best)
        val = jnp.where(m, kc, val)
    o_ref[...] = val


def _quantize(x, keys_t, interpret=False):
    n = x.shape[0]
    blk = 2048
    return pl.pallas_call(
        _quant_body,
        grid=(n // blk,),
        in_specs=[
            pl.BlockSpec((blk, F), lambda i: (i, 0)),
            pl.BlockSpec((NQ, F), lambda i: (0, 0)),
        ],
        out_specs=pl.BlockSpec((blk, F), lambda i: (i, 0)),
        out_shape=jax.ShapeDtypeStruct((n, F), jnp.float32),
        interpret=interpret,
    )(x, keys_t)


def kernel(idxs, feats, ifeats, keys, ikeys, values, scale, _interpret=False):
    i0 = idxs[0].astype(jnp.int32)
    i1 = idxs[1].astype(jnp.int32)

    q0 = _quantize(ifeats[:D0], ikeys[0].T, _interpret) - 0.5
    q1 = _quantize(ifeats[D0:], ikeys[1].T, _interpret) - 0.5
    p0 = _quantize(feats[:D0], keys[0].T, _interpret)
    p1 = _quantize(feats[D0:], keys[1].T, _interpret)

    acc0 = jax.ops.segment_sum(q1[i1], i0, num_segments=D0)
    acc1 = jax.ops.segment_sum(q0[i0], i1, num_segments=D1)
    cnt0 = jnp.bincount(i0, length=D0).astype(jnp.float32)
    cnt1 = jnp.bincount(i1, length=D1).astype(jnp.float32)
    A = p0 + acc0 * jax.lax.rsqrt(cnt0 + 1e-12)[:, None]
    Bt = p1 + acc1 * jax.lax.rsqrt(cnt1 + 1e-12)[:, None]

    w = values[0]
    s = scale[0]
    cAB = w[0] - w[1] - w[2] + w[3]
    cA = w[1] - w[3]
    cB = w[2] - w[3]
    C = jnp.sum(w[3])

    X = (s * cAB)[None, :] * A
    alpha = s * (A @ cA)
    beta = s * (Bt @ cB)

    preds = jnp.sum(X[i0] * Bt[i1], axis=-1) + alpha[i0] + beta[i1] + s * C
    return preds


# speedup vs baseline: 9.7328x; 8.3602x over previous
import functools

import jax
import jax.numpy as jnp
from jax import lax
from jax.experimental import pallas as pl
from jax.experimental.pallas import tpu as pltpu
from jax.experimental.pallas import tpu_sc as plsc

D0 = 16384
D1 = 16384
S = 32768
F = 64
NQ = 16
B = 262144
FA = 80
NCORE = 2
NSUB = 16
CH = 128


def _quant(x_ref, k_ref, o_ref, *, aug, sub_half):
    tf = jax.nn.sigmoid(x_ref[...])
    k = jax.nn.sigmoid(k_ref[...])
    best = jnp.full_like(tf, jnp.inf)
    val = jnp.zeros_like(tf)
    for c in range(NQ):
        kc = k[c][None, :]
        d = jnp.abs(tf - kc)
        m = d < best
        best = jnp.where(m, d, best)
        val = jnp.where(m, kc, val)
    if sub_half:
        val = val - 0.5
    if aug:
        blk = val.shape[0]
        pad = jnp.concatenate(
            [jnp.ones((blk, 1), jnp.float32),
             jnp.zeros((blk, FA - F - 1), jnp.float32)], axis=-1)
        o_ref[...] = jnp.concatenate([val, pad], axis=-1)
    else:
        o_ref[...] = val


def _quantize_aug(x, keys_t):
    n = x.shape[0]
    blk = 2048
    return pl.pallas_call(
        functools.partial(_quant, aug=True, sub_half=True),
        grid=(n // blk,),
        in_specs=[
            pl.BlockSpec((blk, F), lambda i: (i, 0)),
            pl.BlockSpec((NQ, F), lambda i: (0, 0)),
        ],
        out_specs=pl.BlockSpec((blk, FA), lambda i: (i, 0)),
        out_shape=jax.ShapeDtypeStruct((n, FA), jnp.float32),
    )(x, keys_t)


def _k2_call(i0r, i1r, qs0, qs1, zeros):
    ept = B // NSUB
    nch = ept // CH
    rpt = D0 // NSUB
    mesh = plsc.VectorSubcoreMesh(core_axis_name="c", subcore_axis_name="s")

    @functools.partial(
        pl.kernel,
        out_type=[jax.ShapeDtypeStruct((D0, FA), jnp.float32),
                  jax.ShapeDtypeStruct((D1, FA), jnp.float32)],
        mesh=mesh,
        compiler_params=pltpu.CompilerParams(use_tc_tiling_on_sc=False, needs_layout_passes=False),
        scratch_types=[
            pltpu.VMEM((nch, CH), jnp.int32),
            pltpu.VMEM((nch, CH), jnp.int32),
            pltpu.VMEM((CH, FA), jnp.float32),
            pltpu.VMEM_SHARED((D0, FA), jnp.float32),
            pltpu.SemaphoreType.DMA,
        ],
    )
    def k2(i0_hbm, i1_hbm, qs0_hbm, qs1_hbm, z_hbm, acc0_hbm, acc1_hbm,
           sidx, gidx, rows, acc_sh, sem):
        c = lax.axis_index("c")
        s = lax.axis_index("s")
        pltpu.sync_copy(z_hbm.at[pl.ds(s * rpt, rpt)],
                        acc_sh.at[pl.ds(s * rpt, rpt)])

        def side(scat_hbm, gath_hbm, qs_hbm, out_hbm):
            pltpu.sync_copy(scat_hbm.at[pl.ds(s * nch, nch)], sidx)
            pltpu.sync_copy(gath_hbm.at[pl.ds(s * nch, nch)], gidx)
            plsc.subcore_barrier()

            def body(t, carry):
                pltpu.async_copy(qs_hbm.at[gidx.at[t]], rows, sem).wait()
                pltpu.sync_copy(rows, acc_sh.at[sidx.at[t]], add=True)
                return carry
            lax.fori_loop(0, nch, body, 0)
            plsc.subcore_barrier()
            pltpu.sync_copy(acc_sh.at[pl.ds(s * rpt, rpt)],
                            out_hbm.at[pl.ds(s * rpt, rpt)])

        @pl.when(c == 0)
        def _():
            side(i0_hbm, i1_hbm, qs1_hbm, acc0_hbm)

        @pl.when(c == 1)
        def _():
            side(i1_hbm, i0_hbm, qs0_hbm, acc1_hbm)

    return k2(i0r, i1r, qs0, qs1, zeros)


def _k3_body(x_ref, k_ref, acc_ref, c1_ref, c2_ref, c3_ref, xo_ref, ao_ref):
    tf = jax.nn.sigmoid(x_ref[...])
    k = jax.nn.sigmoid(k_ref[...])
    best = jnp.full_like(tf, jnp.inf)
    val = jnp.zeros_like(tf)
    for c in range(NQ):
        kc = k[c][None, :]
        d = jnp.abs(tf - kc)
        m = d < best
        best = jnp.where(m, d, best)
        val = jnp.where(m, kc, val)
    cnt = acc_ref[:, F:F + 1]
    a = val + acc_ref[:, :F] * lax.rsqrt(cnt + 1e-12)
    xo_ref[...] = c1_ref[...] * a
    ao_ref[...] = jnp.sum(a * c2_ref[...], axis=1, keepdims=True) + c3_ref[...]


def _k3_call(feats_h, keys_t, acc, c1, c2, c3):
    blk = 2048
    n = feats_h.shape[0]
    return pl.pallas_call(
        _k3_body,
        grid=(n // blk,),
        in_specs=[
            pl.BlockSpec((blk, F), lambda i: (i, 0)),
            pl.BlockSpec((NQ, F), lambda i: (0, 0)),
            pl.BlockSpec((blk, FA), lambda i: (i, 0)),
            pl.BlockSpec((1, F), lambda i: (0, 0)),
            pl.BlockSpec((1, F), lambda i: (0, 0)),
            pl.BlockSpec((1, 1), lambda i: (0, 0)),
        ],
        out_specs=[
            pl.BlockSpec((blk, F), lambda i: (i, 0)),
            pl.BlockSpec((blk, 1), lambda i: (i, 0)),
        ],
        out_shape=[
            jax.ShapeDtypeStruct((n, F), jnp.float32),
            jax.ShapeDtypeStruct((n, 1), jnp.float32),
        ],
    )(feats_h, keys_t, acc, c1, c2, c3)


def _k4_call(i0r, i1r, X, Y, alpha, beta):
    nw = NCORE * NSUB
    epw = B // nw
    nch = epw // CH
    mesh = plsc.VectorSubcoreMesh(core_axis_name="c", subcore_axis_name="s")

    @functools.partial(
        pl.kernel,
        out_type=jax.ShapeDtypeStruct((B,), jnp.float32),
        mesh=mesh,
        compiler_params=pltpu.CompilerParams(use_tc_tiling_on_sc=False, needs_layout_passes=False),
        scratch_types=[
            pltpu.VMEM((nch, CH), jnp.int32),
            pltpu.VMEM((nch, CH), jnp.int32),
            pltpu.VMEM((CH, F), jnp.float32),
            pltpu.VMEM((CH, F), jnp.float32),
            pltpu.VMEM((CH,), jnp.float32),
            pltpu.VMEM((D0,), jnp.float32),
            pltpu.VMEM((D1,), jnp.float32),
            pltpu.SemaphoreType.DMA,
            pltpu.SemaphoreType.DMA,
        ],
    )
    def k4(i0_hbm, i1_hbm, x_hbm, y_hbm, a_hbm, b_hbm, out_hbm,
           iv, jv, xb, yb, pb, av, bv, sem1, sem2):
        c = lax.axis_index("c")
        s = lax.axis_index("s")
        w = s * NCORE + c
        pltpu.sync_copy(a_hbm, av)
        pltpu.sync_copy(b_hbm, bv)
        pltpu.sync_copy(i0_hbm.at[pl.ds(w * nch, nch)], iv)
        pltpu.sync_copy(i1_hbm.at[pl.ds(w * nch, nch)], jv)

        def chunk(t, carry):
            cp1 = pltpu.async_copy(x_hbm.at[iv.at[t]], xb, sem1)
            cp2 = pltpu.async_copy(y_hbm.at[jv.at[t]], yb, sem2)
            cp1.wait()
            cp2.wait()

            def group(g, carry2):
                rows = g * 16 + lax.iota(jnp.int32, 16)
                acc = jnp.zeros((16,), jnp.float32)
                for f in range(F):
                    cols = jnp.full((16,), f, jnp.int32)
                    xv = plsc.load_gather(xb, [rows, cols])
                    yv = plsc.load_gather(yb, [rows, cols])
                    acc = acc + xv * yv
                tt = jnp.full((16,), t, jnp.int32)
                ia = plsc.load_gather(iv, [tt, rows])
                ja = plsc.load_gather(jv, [tt, rows])
                acc = acc + plsc.load_gather(av, [ia]) + plsc.load_gather(bv, [ja])
                pb[pl.ds(g * 16, 16)] = acc
                return carry2
            lax.fori_loop(0, CH // 16, group, 0)
            pltpu.sync_copy(pb, out_hbm.at[pl.ds(w * epw + t * CH, CH)])
            return carry
        lax.fori_loop(0, nch, chunk, 0)

    return k4(i0r, i1r, X, Y, alpha, beta)


def kernel(idxs, feats, ifeats, keys, ikeys, values, scale):
    i0 = idxs[0].astype(jnp.int32)
    i1 = idxs[1].astype(jnp.int32)
    i0r = i0.reshape(B // CH, CH)
    i1r = i1.reshape(B // CH, CH)

    qs0 = _quantize_aug(ifeats[:D0], ikeys[0].T)
    qs1 = _quantize_aug(ifeats[D0:], ikeys[1].T)

    zeros = jnp.zeros((D0, FA), jnp.float32)
    acc0, acc1 = _k2_call(i0r, i1r, qs0, qs1, zeros)

    w = values[0]
    s = scale[0]
    cAB = w[0] - w[1] - w[2] + w[3]
    cA = w[1] - w[3]
    cB = w[2] - w[3]
    C = jnp.sum(w[3])

    X, alpha = _k3_call(feats[:D0], keys[0].T, acc0,
                        (s * cAB)[None, :], (s * cA)[None, :],
                        (s * C)[None, None])
    Y, beta = _k3_call(feats[D0:], keys[1].T, acc1,
                       jnp.ones((1, F), jnp.float32), (s * cB)[None, :],
                       jnp.zeros((1, 1), jnp.float32))
    alpha = alpha[:, 0]
    beta = beta[:, 0]
    preds = _k4_call(i0r, i1r, X, Y, alpha, beta)
    return preds


# speedup vs baseline: 11.5188x; 1.1835x over previous
import functools

import jax
import jax.numpy as jnp
from jax import lax
from jax.experimental import pallas as pl
from jax.experimental.pallas import tpu as pltpu
from jax.experimental.pallas import tpu_sc as plsc

D0 = 16384
D1 = 16384
S = 32768
F = 64
NQ = 16
B = 262144
FA = 72
NCORE = 2
NSUB = 16
CH = 128


def _quant(x_ref, k_ref, o_ref, *, aug, sub_half):
    tf = jax.nn.sigmoid(x_ref[...])
    k = jax.nn.sigmoid(k_ref[...])
    best = jnp.full_like(tf, jnp.inf)
    val = jnp.zeros_like(tf)
    for c in range(NQ):
        kc = k[c][None, :]
        d = jnp.abs(tf - kc)
        m = d < best
        best = jnp.where(m, d, best)
        val = jnp.where(m, kc, val)
    if sub_half:
        val = val - 0.5
    if aug:
        blk = val.shape[0]
        pad = jnp.concatenate(
            [jnp.ones((blk, 1), jnp.float32),
             jnp.zeros((blk, FA - F - 1), jnp.float32)], axis=-1)
        o_ref[...] = jnp.concatenate([val, pad], axis=-1)
    else:
        o_ref[...] = val


def _quantize_aug(x, keys_t):
    n = x.shape[0]
    blk = 2048
    return pl.pallas_call(
        functools.partial(_quant, aug=True, sub_half=True),
        grid=(n // blk,),
        in_specs=[
            pl.BlockSpec((blk, F), lambda i: (i, 0)),
            pl.BlockSpec((NQ, F), lambda i: (0, 0)),
        ],
        out_specs=pl.BlockSpec((blk, FA), lambda i: (i, 0)),
        out_shape=jax.ShapeDtypeStruct((n, FA), jnp.float32),
    )(x, keys_t)


def _k2_call(i0r, i1r, qs0, qs1, zeros):
    ept = B // NSUB
    nch = ept // CH
    rpt = D0 // NSUB
    mesh = plsc.VectorSubcoreMesh(core_axis_name="c", subcore_axis_name="s")

    @functools.partial(
        pl.kernel,
        out_type=[jax.ShapeDtypeStruct((D0, FA), jnp.float32),
                  jax.ShapeDtypeStruct((D1, FA), jnp.float32)],
        mesh=mesh,
        compiler_params=pltpu.CompilerParams(use_tc_tiling_on_sc=False, needs_layout_passes=False),
        scratch_types=[
            pltpu.VMEM((nch, CH), jnp.int32),
            pltpu.VMEM((nch, CH), jnp.int32),
            pltpu.VMEM((CH, FA), jnp.float32),
            pltpu.VMEM((CH, FA), jnp.float32),
            pltpu.VMEM_SHARED((D0, FA), jnp.float32),
            pltpu.SemaphoreType.DMA,
            pltpu.SemaphoreType.DMA,
        ],
    )
    def k2(i0_hbm, i1_hbm, qs0_hbm, qs1_hbm, z_hbm, acc0_hbm, acc1_hbm,
           sidx, gidx, rows0, rows1, acc_sh, sem0, sem1):
        c = lax.axis_index("c")
        s = lax.axis_index("s")
        pltpu.sync_copy(z_hbm.at[pl.ds(s * rpt, rpt)],
                        acc_sh.at[pl.ds(s * rpt, rpt)])

        def side(scat_hbm, gath_hbm, qs_hbm, out_hbm):
            pltpu.sync_copy(scat_hbm.at[pl.ds(s * nch, nch)], sidx)
            pltpu.sync_copy(gath_hbm.at[pl.ds(s * nch, nch)], gidx)
            plsc.subcore_barrier()

            def fire(t, buf, sem):
                pltpu.async_copy(qs_hbm.at[gidx.at[t]], buf, sem)

            def drain(buf, sem):
                pltpu.make_async_copy(qs_hbm.at[gidx.at[0]], buf, sem).wait()

            fire(0, rows0, sem0)

            def body(u, carry):
                t0 = 2 * u
                t1 = t0 + 1
                fire(t1, rows1, sem1)
                drain(rows0, sem0)
                pltpu.sync_copy(rows0, acc_sh.at[sidx.at[t0]], add=True)

                @pl.when(t1 + 1 < nch)
                def _():
                    fire(t1 + 1, rows0, sem0)
                drain(rows1, sem1)
                pltpu.sync_copy(rows1, acc_sh.at[sidx.at[t1]], add=True)
                return carry
            lax.fori_loop(0, nch // 2, body, 0)
            plsc.subcore_barrier()
            pltpu.sync_copy(acc_sh.at[pl.ds(s * rpt, rpt)],
                            out_hbm.at[pl.ds(s * rpt, rpt)])

        @pl.when(c == 0)
        def _():
            side(i0_hbm, i1_hbm, qs1_hbm, acc0_hbm)

        @pl.when(c == 1)
        def _():
            side(i1_hbm, i0_hbm, qs0_hbm, acc1_hbm)

    return k2(i0r, i1r, qs0, qs1, zeros)


def _k3_body(x_ref, k_ref, acc_ref, c1_ref, c2_ref, c3_ref, xo_ref, ao_ref):
    tf = jax.nn.sigmoid(x_ref[...])
    k = jax.nn.sigmoid(k_ref[...])
    best = jnp.full_like(tf, jnp.inf)
    val = jnp.zeros_like(tf)
    for c in range(NQ):
        kc = k[c][None, :]
        d = jnp.abs(tf - kc)
        m = d < best
        best = jnp.where(m, d, best)
        val = jnp.where(m, kc, val)
    cnt = acc_ref[:, F:F + 1]
    a = val + acc_ref[:, :F] * lax.rsqrt(cnt + 1e-12)
    xo_ref[...] = c1_ref[...] * a
    ao_ref[...] = jnp.sum(a * c2_ref[...], axis=1, keepdims=True) + c3_ref[...]


def _k3_call(feats_h, keys_t, acc, c1, c2, c3):
    blk = 2048
    n = feats_h.shape[0]
    return pl.pallas_call(
        _k3_body,
        grid=(n // blk,),
        in_specs=[
            pl.BlockSpec((blk, F), lambda i: (i, 0)),
            pl.BlockSpec((NQ, F), lambda i: (0, 0)),
            pl.BlockSpec((blk, FA), lambda i: (i, 0)),
            pl.BlockSpec((1, F), lambda i: (0, 0)),
            pl.BlockSpec((1, F), lambda i: (0, 0)),
            pl.BlockSpec((1, 1), lambda i: (0, 0)),
        ],
        out_specs=[
            pl.BlockSpec((blk, F), lambda i: (i, 0)),
            pl.BlockSpec((blk, 1), lambda i: (i, 0)),
        ],
        out_shape=[
            jax.ShapeDtypeStruct((n, F), jnp.float32),
            jax.ShapeDtypeStruct((n, 1), jnp.float32),
        ],
    )(feats_h, keys_t, acc, c1, c2, c3)


def _k4_call(i0r, i1r, X, Y, alpha, beta):
    nw = NCORE * NSUB
    epw = B // nw
    nch = epw // CH
    mesh = plsc.VectorSubcoreMesh(core_axis_name="c", subcore_axis_name="s")

    @functools.partial(
        pl.kernel,
        out_type=jax.ShapeDtypeStruct((B // CH, CH), jnp.float32),
        mesh=mesh,
        compiler_params=pltpu.CompilerParams(use_tc_tiling_on_sc=False, needs_layout_passes=False),
        scratch_types=[
            pltpu.VMEM((nch, CH), jnp.int32),
            pltpu.VMEM((nch, CH), jnp.int32),
            pltpu.VMEM((CH, F), jnp.float32),
            pltpu.VMEM((CH, F), jnp.float32),
            pltpu.VMEM((CH, F), jnp.float32),
            pltpu.VMEM((CH, F), jnp.float32),
            pltpu.VMEM((nch, CH), jnp.float32),
            pltpu.VMEM((D0,), jnp.float32),
            pltpu.VMEM((D1,), jnp.float32),
            pltpu.SemaphoreType.DMA,
            pltpu.SemaphoreType.DMA,
        ],
    )
    def k4(i0_hbm, i1_hbm, x_hbm, y_hbm, a_hbm, b_hbm, out_hbm,
           iv, jv, xb0, yb0, xb1, yb1, pb, av, bv, sem0, sem1):
        c = lax.axis_index("c")
        s = lax.axis_index("s")
        w = s * NCORE + c
        pltpu.sync_copy(a_hbm, av)
        pltpu.sync_copy(b_hbm, bv)
        pltpu.sync_copy(i0_hbm.at[pl.ds(w * nch, nch)], iv)
        pltpu.sync_copy(i1_hbm.at[pl.ds(w * nch, nch)], jv)

        def fire(t, xbuf, ybuf, sem):
            pltpu.async_copy(x_hbm.at[iv.at[t]], xbuf, sem)
            pltpu.async_copy(y_hbm.at[jv.at[t]], ybuf, sem)

        def drain(xbuf, ybuf, sem):
            pltpu.make_async_copy(x_hbm.at[iv.at[0]], xbuf, sem).wait()
            pltpu.make_async_copy(y_hbm.at[jv.at[0]], ybuf, sem).wait()

        def compute(t, xbuf, ybuf):
            def group(g, carry2):
                rows = g * 16 + lax.iota(jnp.int32, 16)
                acc = jnp.zeros((16,), jnp.float32)
                for f in range(F):
                    cols = jnp.full((16,), f, jnp.int32)
                    xv = plsc.load_gather(xbuf, [rows, cols])
                    yv = plsc.load_gather(ybuf, [rows, cols])
                    acc = acc + xv * yv
                tt = jnp.full((16,), t, jnp.int32)
                ia = plsc.load_gather(iv, [tt, rows])
                ja = plsc.load_gather(jv, [tt, rows])
                acc = acc + plsc.load_gather(av, [ia]) + plsc.load_gather(bv, [ja])
                plsc.store_scatter(pb, [tt, rows], acc)
                return carry2
            lax.fori_loop(0, CH // 16, group, 0)

        fire(0, xb0, yb0, sem0)

        def pair(u, carry):
            t0 = 2 * u
            t1 = t0 + 1
            fire(t1, xb1, yb1, sem1)
            drain(xb0, yb0, sem0)
            compute(t0, xb0, yb0)

            @pl.when(t1 + 1 < nch)
            def _():
                fire(t1 + 1, xb0, yb0, sem0)
            drain(xb1, yb1, sem1)
            compute(t1, xb1, yb1)
            return carry
        lax.fori_loop(0, nch // 2, pair, 0)
        pltpu.sync_copy(pb, out_hbm.at[pl.ds(w * nch, nch)])

    return k4(i0r, i1r, X, Y, alpha, beta).reshape(B)


def kernel(idxs, feats, ifeats, keys, ikeys, values, scale):
    i0 = idxs[0].astype(jnp.int32)
    i1 = idxs[1].astype(jnp.int32)
    i0r = i0.reshape(B // CH, CH)
    i1r = i1.reshape(B // CH, CH)

    qs0 = _quantize_aug(ifeats[:D0], ikeys[0].T)
    qs1 = _quantize_aug(ifeats[D0:], ikeys[1].T)

    zeros = jnp.zeros((D0, FA), jnp.float32)
    acc0, acc1 = _k2_call(i0r, i1r, qs0, qs1, zeros)

    w = values[0]
    s = scale[0]
    cAB = w[0] - w[1] - w[2] + w[3]
    cA = w[1] - w[3]
    cB = w[2] - w[3]
    C = jnp.sum(w[3])

    X, alpha = _k3_call(feats[:D0], keys[0].T, acc0,
                        (s * cAB)[None, :], (s * cA)[None, :],
                        (s * C)[None, None])
    Y, beta = _k3_call(feats[D0:], keys[1].T, acc1,
                       jnp.ones((1, F), jnp.float32), (s * cB)[None, :],
                       jnp.zeros((1, 1), jnp.float32))
    alpha = alpha[:, 0]
    beta = beta[:, 0]
    preds = _k4_call(i0r, i1r, X, Y, alpha, beta)
    return preds


# speedup vs baseline: 25.1710x; 2.1852x over previous
import functools

import jax
import jax.numpy as jnp
from jax import lax
from jax.experimental import pallas as pl
from jax.experimental.pallas import tpu as pltpu
from jax.experimental.pallas import tpu_sc as plsc

D0 = 16384
D1 = 16384
S = 32768
F = 64
NQ = 16
B = 262144
FA = 72
NCORE = 2
NSUB = 16
CH = 128


SUB = 128


def _quant_tile(x_tile, k):
    tf = jax.nn.sigmoid(x_tile)
    best = jnp.full_like(tf, jnp.inf)
    val = jnp.zeros_like(tf)
    for c in range(NQ):
        kc = k[c][None, :]
        d = jnp.abs(tf - kc)
        m = d < best
        best = jnp.where(m, d, best)
        val = jnp.where(m, kc, val)
    return val


def _quant(x_ref, k_ref, o_ref, *, aug, sub_half):
    k = jax.nn.sigmoid(k_ref[...])
    blk = x_ref.shape[0]

    for r in range(blk // SUB):
        sl = pl.ds(r * SUB, SUB)
        val = _quant_tile(x_ref[sl, :], k)
        if sub_half:
            val = val - 0.5
        if aug:
            pad = jnp.concatenate(
                [jnp.ones((SUB, 1), jnp.float32),
                 jnp.zeros((SUB, FA - F - 1), jnp.float32)], axis=-1)
            o_ref[sl, :] = jnp.concatenate([val, pad], axis=-1)
        else:
            o_ref[sl, :] = val


def _quantize_aug(x, keys_t):
    n = x.shape[0]
    blk = 2048
    return pl.pallas_call(
        functools.partial(_quant, aug=True, sub_half=True),
        grid=(n // blk,),
        in_specs=[
            pl.BlockSpec((blk, F), lambda i: (i, 0)),
            pl.BlockSpec((NQ, F), lambda i: (0, 0)),
        ],
        out_specs=pl.BlockSpec((blk, FA), lambda i: (i, 0)),
        out_shape=jax.ShapeDtypeStruct((n, FA), jnp.float32),
    )(x, keys_t)


def _quantize_plain(x, keys_t):
    n = x.shape[0]
    blk = 2048
    return pl.pallas_call(
        functools.partial(_quant, aug=False, sub_half=False),
        grid=(n // blk,),
        in_specs=[
            pl.BlockSpec((blk, F), lambda i: (i, 0)),
            pl.BlockSpec((NQ, F), lambda i: (0, 0)),
        ],
        out_specs=pl.BlockSpec((blk, F), lambda i: (i, 0)),
        out_shape=jax.ShapeDtypeStruct((n, F), jnp.float32),
    )(x, keys_t)


def _k2_call(i0r, i1r, qs0, qs1, zeros):
    ept = B // NSUB
    nch = ept // CH
    nh = nch // 2
    rpt = D0 // NSUB
    mesh = plsc.VectorSubcoreMesh(core_axis_name="c", subcore_axis_name="s")

    @functools.partial(
        pl.kernel,
        out_type=[jax.ShapeDtypeStruct((D0, FA), jnp.float32),
                  jax.ShapeDtypeStruct((D1, FA), jnp.float32)],
        mesh=mesh,
        compiler_params=pltpu.CompilerParams(use_tc_tiling_on_sc=False, needs_layout_passes=False),
        scratch_types=[
            pltpu.VMEM((nh, CH), jnp.int32),
            pltpu.VMEM((nh, CH), jnp.int32),
            pltpu.VMEM((CH, FA), jnp.float32),
            pltpu.VMEM((CH, FA), jnp.float32),
            pltpu.VMEM((CH, FA), jnp.float32),
            pltpu.VMEM((CH, FA), jnp.float32),
            pltpu.VMEM_SHARED((D0, FA), jnp.float32),
            pltpu.SemaphoreType.DMA,
            pltpu.SemaphoreType.DMA,
            pltpu.SemaphoreType.DMA,
            pltpu.SemaphoreType.DMA,
        ],
    )
    def k2(i0_hbm, i1_hbm, qs0_hbm, qs1_hbm, z_hbm, acc0_hbm, acc1_hbm,
           sidx, gidx, rows0, rows1, rows2, rows3, acc_sh,
           sem0, sem1, sem2, sem3):
        c = lax.axis_index("c")
        s = lax.axis_index("s")
        def zb(r, carry):
            pltpu.sync_copy(z_hbm, acc_sh.at[pl.ds(s * rpt + r * CH, CH)])
            return carry
        lax.fori_loop(0, rpt // CH, zb, 0)

        def side(scat_hbm, gath_hbm, qs_hbm, out_hbm):
            slots = [(rows0, sem0), (rows1, sem1), (rows2, sem2), (rows3, sem3)]

            def load_idx(h):
                off = s * nch + h * nh
                pltpu.sync_copy(scat_hbm.at[pl.ds(off, nh)], sidx)
                pltpu.sync_copy(gath_hbm.at[pl.ds(off, nh)], gidx)

            def fire(t, buf, sem):
                pltpu.async_copy(qs_hbm.at[gidx.at[t]], buf, sem)

            def drain(buf, sem):
                pltpu.make_async_copy(qs_hbm.at[gidx.at[0]], buf, sem).wait()

            load_idx(0)
            plsc.subcore_barrier()

            def half(h, carry):
                for k in range(3):
                    fire(k, *slots[k])

                def body(u, carry2):
                    for k in range(4):
                        t = 4 * u + k
                        buf, sem = slots[k]
                        drain(buf, sem)
                        pltpu.sync_copy(buf, acc_sh.at[sidx.at[t]], add=True)

                        @pl.when(t + 3 < nh)
                        def _():
                            fire(t + 3, *slots[(k + 3) % 4])
                    return carry2
                lax.fori_loop(0, nh // 4, body, 0)

                @pl.when(h == 0)
                def _():
                    load_idx(1)
                return carry
            lax.fori_loop(0, 2, half, 0)
            plsc.subcore_barrier()
            pltpu.sync_copy(acc_sh.at[pl.ds(s * rpt, rpt)],
                            out_hbm.at[pl.ds(s * rpt, rpt)])

        @pl.when(c == 0)
        def _():
            side(i0_hbm, i1_hbm, qs1_hbm, acc0_hbm)

        @pl.when(c == 1)
        def _():
            side(i1_hbm, i0_hbm, qs0_hbm, acc1_hbm)

    return k2(i0r, i1r, qs0, qs1, zeros)


def _k3_body(p_ref, acc_ref, c1_ref, c2_ref, c3_ref, xo_ref, ao_ref):
    blk = p_ref.shape[0]

    for r in range(blk // SUB):
        sl = pl.ds(r * SUB, SUB)
        cnt = acc_ref[sl, F:F + 1]
        a = p_ref[sl, :] + acc_ref[sl, :F] * lax.rsqrt(cnt + 1e-12)
        xo_ref[sl, :] = c1_ref[...] * a
        ao_ref[sl, :] = (jnp.sum(a * c2_ref[...], axis=1, keepdims=True)
                         + c3_ref[...])


def _k3_call(p_h, acc, c1, c2, c3):
    blk = 2048
    n = p_h.shape[0]
    return pl.pallas_call(
        _k3_body,
        grid=(n // blk,),
        in_specs=[
            pl.BlockSpec((blk, F), lambda i: (i, 0)),
            pl.BlockSpec((blk, FA), lambda i: (i, 0)),
            pl.BlockSpec((1, F), lambda i: (0, 0)),
            pl.BlockSpec((1, F), lambda i: (0, 0)),
            pl.BlockSpec((1, 1), lambda i: (0, 0)),
        ],
        out_specs=[
            pl.BlockSpec((blk, F), lambda i: (i, 0)),
            pl.BlockSpec((blk, 1), lambda i: (i, 0)),
        ],
        out_shape=[
            jax.ShapeDtypeStruct((n, F), jnp.float32),
            jax.ShapeDtypeStruct((n, 1), jnp.float32),
        ],
    )(p_h, acc, c1, c2, c3)


def _k4_call(i0r, i1r, X, Y, alpha, beta):
    nw = NCORE * NSUB
    epw = B // nw
    nch = epw // CH
    mesh = plsc.VectorSubcoreMesh(core_axis_name="c", subcore_axis_name="s")

    @functools.partial(
        pl.kernel,
        out_type=jax.ShapeDtypeStruct((B // CH, CH), jnp.float32),
        mesh=mesh,
        compiler_params=pltpu.CompilerParams(use_tc_tiling_on_sc=False, needs_layout_passes=False),
        scratch_types=[
            pltpu.VMEM((nch, CH), jnp.int32),
            pltpu.VMEM((nch, CH), jnp.int32),
            pltpu.VMEM((CH, F), jnp.float32),
            pltpu.VMEM((CH, F), jnp.float32),
            pltpu.VMEM((CH, F), jnp.float32),
            pltpu.VMEM((CH, F), jnp.float32),
            pltpu.VMEM((nch, CH), jnp.float32),
            pltpu.VMEM((D0,), jnp.float32),
            pltpu.VMEM((D1,), jnp.float32),
            pltpu.SemaphoreType.DMA,
            pltpu.SemaphoreType.DMA,
        ],
    )
    def k4(i0_hbm, i1_hbm, x_hbm, y_hbm, a_hbm, b_hbm, out_hbm,
           iv, jv, xb0, yb0, xb1, yb1, pb, av, bv, sem0, sem1):
        c = lax.axis_index("c")
        s = lax.axis_index("s")
        w = s * NCORE + c
        pltpu.sync_copy(a_hbm, av)
        pltpu.sync_copy(b_hbm, bv)
        pltpu.sync_copy(i0_hbm.at[pl.ds(w * nch, nch)], iv)
        pltpu.sync_copy(i1_hbm.at[pl.ds(w * nch, nch)], jv)

        def fire(t, xbuf, ybuf, sem):
            pltpu.async_copy(x_hbm.at[iv.at[t]], xbuf, sem)
            pltpu.async_copy(y_hbm.at[jv.at[t]], ybuf, sem)

        def drain(xbuf, ybuf, sem):
            pltpu.make_async_copy(x_hbm.at[iv.at[0]], xbuf, sem).wait()
            pltpu.make_async_copy(y_hbm.at[jv.at[0]], ybuf, sem).wait()

        def compute(t, xbuf, ybuf):
            def group(g, carry2):
                lane = lax.iota(jnp.int32, 16)
                rows = g * 16 + lane
                acc = jnp.zeros((16,), jnp.float32)
                for f in range(F):
                    cols = jnp.bitwise_and(lane + f, F - 1)
                    xv = plsc.load_gather(xbuf, [rows, cols])
                    yv = plsc.load_gather(ybuf, [rows, cols])
                    acc = acc + xv * yv
                tt = jnp.full((16,), t, jnp.int32)
                ia = plsc.load_gather(iv, [tt, rows])
                ja = plsc.load_gather(jv, [tt, rows])
                acc = acc + plsc.load_gather(av, [ia]) + plsc.load_gather(bv, [ja])
                plsc.store_scatter(pb, [tt, rows], acc)
                return carry2
            lax.fori_loop(0, CH // 16, group, 0)

        fire(0, xb0, yb0, sem0)

        def pair(u, carry):
            t0 = 2 * u
            t1 = t0 + 1
            fire(t1, xb1, yb1, sem1)
            drain(xb0, yb0, sem0)
            compute(t0, xb0, yb0)

            @pl.when(t1 + 1 < nch)
            def _():
                fire(t1 + 1, xb0, yb0, sem0)
            drain(xb1, yb1, sem1)
            compute(t1, xb1, yb1)
            return carry
        lax.fori_loop(0, nch // 2, pair, 0)
        pltpu.sync_copy(pb, out_hbm.at[pl.ds(w * nch, nch)])

    return k4(i0r, i1r, X, Y, alpha, beta).reshape(B)


def kernel(idxs, feats, ifeats, keys, ikeys, values, scale):
    i0 = idxs[0].astype(jnp.int32)
    i1 = idxs[1].astype(jnp.int32)
    i0r = i0.reshape(B // CH, CH)
    i1r = i1.reshape(B // CH, CH)

    qs0 = _quantize_aug(ifeats[:D0], ikeys[0].T)
    qs1 = _quantize_aug(ifeats[D0:], ikeys[1].T)

    zeros = jnp.zeros((CH, FA), jnp.float32)
    acc0, acc1 = _k2_call(i0r, i1r, qs0, qs1, zeros)
    p0 = _quantize_plain(feats[:D0], keys[0].T)
    p1 = _quantize_plain(feats[D0:], keys[1].T)

    w = values[0]
    s = scale[0]
    cAB = w[0] - w[1] - w[2] + w[3]
    cA = w[1] - w[3]
    cB = w[2] - w[3]
    C = jnp.sum(w[3])

    X, alpha = _k3_call(p0, acc0,
                        (s * cAB)[None, :], (s * cA)[None, :],
                        (s * C)[None, None])
    Y, beta = _k3_call(p1, acc1,
                       jnp.ones((1, F), jnp.float32), (s * cB)[None, :],
                       jnp.zeros((1, 1), jnp.float32))
    preds = _k4_call(i0r, i1r, X, Y, alpha[:, 0], beta[:, 0])
    return preds


# speedup vs baseline: 25.9666x; 1.0316x over previous
import functools

import jax
import jax.numpy as jnp
from jax import lax
from jax.experimental import pallas as pl
from jax.experimental.pallas import tpu as pltpu
from jax.experimental.pallas import tpu_sc as plsc

D0 = 16384
D1 = 16384
S = 32768
F = 64
NQ = 16
B = 262144
FA = 72
NCORE = 2
NSUB = 16
CH = 128
SUB = 128

_SC_PARAMS = pltpu.CompilerParams(use_tc_tiling_on_sc=False,
                                  needs_layout_passes=False)


def _quant_tile(x_tile, k):
    tf = jax.nn.sigmoid(x_tile)
    best = jnp.full_like(tf, jnp.inf)
    val = jnp.zeros_like(tf)
    for c in range(NQ):
        kc = k[c][None, :]
        d = jnp.abs(tf - kc)
        m = d < best
        best = jnp.where(m, d, best)
        val = jnp.where(m, kc, val)
    return val


def _quant(x_ref, k_ref, o_ref, *, aug):
    k = jax.nn.sigmoid(k_ref[0])
    blk = x_ref.shape[0]

    for r in range(blk // SUB):
        sl = pl.ds(r * SUB, SUB)
        val = _quant_tile(x_ref[sl, :], k)
        if aug:
            pad = jnp.concatenate(
                [jnp.ones((SUB, 1), jnp.float32),
                 jnp.zeros((SUB, FA - F - 1), jnp.float32)], axis=-1)
            o_ref[sl, :] = jnp.concatenate([val - 0.5, pad], axis=-1)
        else:
            o_ref[sl, :] = val


def _quantize(x, keys_t2, aug):
    blk = 2048
    nb_half = D0 // blk
    fo = FA if aug else F
    return pl.pallas_call(
        functools.partial(_quant, aug=aug),
        grid=(S // blk,),
        in_specs=[
            pl.BlockSpec((blk, F), lambda i: (i, 0)),
            pl.BlockSpec((1, NQ, F), lambda i: (i // nb_half, 0, 0)),
        ],
        out_specs=pl.BlockSpec((blk, fo), lambda i: (i, 0)),
        out_shape=jax.ShapeDtypeStruct((S, fo), jnp.float32),
    )(x, keys_t2)


def _k2_call(i0r, i1r, i1or, qs, zeros):
    ept = B // NSUB
    nch = ept // CH
    nh = nch // 2
    rpt = D0 // NSUB
    mesh = plsc.VectorSubcoreMesh(core_axis_name="c", subcore_axis_name="s")

    @functools.partial(
        pl.kernel,
        out_type=jax.ShapeDtypeStruct((S, FA), jnp.float32),
        mesh=mesh,
        compiler_params=_SC_PARAMS,
        scratch_types=[
            pltpu.VMEM((nh, CH), jnp.int32),
            pltpu.VMEM((nh, CH), jnp.int32),
            pltpu.VMEM((CH, FA), jnp.float32),
            pltpu.VMEM((CH, FA), jnp.float32),
            pltpu.VMEM((CH, FA), jnp.float32),
            pltpu.VMEM((CH, FA), jnp.float32),
            pltpu.VMEM_SHARED((D0, FA), jnp.float32),
            pltpu.SemaphoreType.DMA,
            pltpu.SemaphoreType.DMA,
            pltpu.SemaphoreType.DMA,
            pltpu.SemaphoreType.DMA,
        ],
    )
    def k2(i0_hbm, i1_hbm, i1o_hbm, qs_hbm, z_hbm, acc_hbm,
           sidx, gidx, rows0, rows1, rows2, rows3, acc_sh,
           sem0, sem1, sem2, sem3):
        c = lax.axis_index("c")
        s = lax.axis_index("s")

        def zb(r, carry):
            pltpu.sync_copy(z_hbm, acc_sh.at[pl.ds(s * rpt + r * CH, CH)])
            return carry
        lax.fori_loop(0, rpt // CH, zb, 0)

        def side(scat_hbm, gath_hbm):
            slots = [(rows0, sem0), (rows1, sem1), (rows2, sem2), (rows3, sem3)]

            def load_idx(h):
                off = s * nch + h * nh
                pltpu.sync_copy(scat_hbm.at[pl.ds(off, nh)], sidx)
                pltpu.sync_copy(gath_hbm.at[pl.ds(off, nh)], gidx)

            def fire(t, buf, sem):
                pltpu.async_copy(qs_hbm.at[gidx.at[t]], buf, sem)

            def drain(buf, sem):
                pltpu.make_async_copy(qs_hbm.at[gidx.at[0]], buf, sem).wait()

            load_idx(0)
            plsc.subcore_barrier()

            def half(h, carry):
                for k in range(3):
                    fire(k, *slots[k])

                def body(u, carry2):
                    for k in range(4):
                        t = 4 * u + k
                        buf, sem = slots[k]
                        drain(buf, sem)
                        pltpu.sync_copy(buf, acc_sh.at[sidx.at[t]], add=True)

                        @pl.when(t + 3 < nh)
                        def _():
                            fire(t + 3, *slots[(k + 3) % 4])
                    return carry2
                lax.fori_loop(0, nh // 4, body, 0)

                @pl.when(h == 0)
                def _():
                    load_idx(1)
                return carry
            lax.fori_loop(0, 2, half, 0)
            plsc.subcore_barrier()
            pltpu.sync_copy(acc_sh.at[pl.ds(s * rpt, rpt)],
                            acc_hbm.at[pl.ds(c * D0 + s * rpt, rpt)])

        @pl.when(c == 0)
        def _():
            side(i0_hbm, i1o_hbm)

        @pl.when(c == 1)
        def _():
            side(i1_hbm, i0_hbm)

    return k2(i0r, i1r, i1or, qs, zeros)


def _k3_body(p_ref, acc_ref, c1_ref, c2_ref, c3_ref, xo_ref, ao_ref):
    blk = p_ref.shape[0]

    for r in range(blk // SUB):
        sl = pl.ds(r * SUB, SUB)
        cnt = acc_ref[sl, F:F + 1]
        a = p_ref[sl, :] + acc_ref[sl, :F] * lax.rsqrt(cnt + 1e-12)
        xo_ref[sl, :] = c1_ref[0] * a
        ao_ref[sl, :] = (jnp.sum(a * c2_ref[0], axis=1, keepdims=True)
                         + c3_ref[0])


def _k3_call(p, acc, c1, c2, c3):
    blk = 2048
    nb_half = D0 // blk
    return pl.pallas_call(
        _k3_body,
        grid=(S // blk,),
        in_specs=[
            pl.BlockSpec((blk, F), lambda i: (i, 0)),
            pl.BlockSpec((blk, FA), lambda i: (i, 0)),
            pl.BlockSpec((1, 1, F), lambda i: (i // nb_half, 0, 0)),
            pl.BlockSpec((1, 1, F), lambda i: (i // nb_half, 0, 0)),
            pl.BlockSpec((1, 1, 1), lambda i: (i // nb_half, 0, 0)),
        ],
        out_specs=[
            pl.BlockSpec((blk, F), lambda i: (i, 0)),
            pl.BlockSpec((blk, 1), lambda i: (i, 0)),
        ],
        out_shape=[
            jax.ShapeDtypeStruct((S, F), jnp.float32),
            jax.ShapeDtypeStruct((S, 1), jnp.float32),
        ],
    )(p, acc, c1, c2, c3)


def _k4_call(i0r, i1or, X, ab):
    nw = NCORE * NSUB
    epw = B // nw
    nch = epw // CH
    mesh = plsc.VectorSubcoreMesh(core_axis_name="c", subcore_axis_name="s")

    @functools.partial(
        pl.kernel,
        out_type=jax.ShapeDtypeStruct((B // CH, CH), jnp.float32),
        mesh=mesh,
        compiler_params=_SC_PARAMS,
        scratch_types=[
            pltpu.VMEM((nch, CH), jnp.int32),
            pltpu.VMEM((nch, CH), jnp.int32),
            pltpu.VMEM((CH, F), jnp.float32),
            pltpu.VMEM((CH, F), jnp.float32),
            pltpu.VMEM((CH, F), jnp.float32),
            pltpu.VMEM((CH, F), jnp.float32),
            pltpu.VMEM((nch, CH), jnp.float32),
            pltpu.VMEM((S,), jnp.float32),
            pltpu.SemaphoreType.DMA,
            pltpu.SemaphoreType.DMA,
        ],
    )
    def k4(i0_hbm, i1o_hbm, x_hbm, ab_hbm, out_hbm,
           iv, jv, xb0, yb0, xb1, yb1, pb, abv, sem0, sem1):
        c = lax.axis_index("c")
        s = lax.axis_index("s")
        w = s * NCORE + c
        pltpu.sync_copy(ab_hbm, abv)
        pltpu.sync_copy(i0_hbm.at[pl.ds(w * nch, nch)], iv)
        pltpu.sync_copy(i1o_hbm.at[pl.ds(w * nch, nch)], jv)

        def fire(t, xbuf, ybuf, sem):
            pltpu.async_copy(x_hbm.at[iv.at[t]], xbuf, sem)
            pltpu.async_copy(x_hbm.at[jv.at[t]], ybuf, sem)

        def drain(xbuf, ybuf, sem):
            pltpu.make_async_copy(x_hbm.at[iv.at[0]], xbuf, sem).wait()
            pltpu.make_async_copy(x_hbm.at[jv.at[0]], ybuf, sem).wait()

        def compute(t, xbuf, ybuf):
            def group(g, carry2):
                lane = lax.iota(jnp.int32, 16)
                rows = g * 16 + lane
                acc = jnp.zeros((16,), jnp.float32)
                for f in range(F):
                    cols = jnp.bitwise_and(lane + f, F - 1)
                    xv = plsc.load_gather(xbuf, [rows, cols])
                    yv = plsc.load_gather(ybuf, [rows, cols])
                    acc = acc + xv * yv
                tt = jnp.full((16,), t, jnp.int32)
                ia = plsc.load_gather(iv, [tt, rows])
                ja = plsc.load_gather(jv, [tt, rows])
                acc = acc + plsc.load_gather(abv, [ia]) + plsc.load_gather(abv, [ja])
                plsc.store_scatter(pb, [tt, rows], acc)
                return carry2
            lax.fori_loop(0, CH // 16, group, 0)

        fire(0, xb0, yb0, sem0)

        def pair(u, carry):
            t0 = 2 * u
            t1 = t0 + 1
            fire(t1, xb1, yb1, sem1)
            drain(xb0, yb0, sem0)
            compute(t0, xb0, yb0)

            @pl.when(t1 + 1 < nch)
            def _():
                fire(t1 + 1, xb0, yb0, sem0)
            drain(xb1, yb1, sem1)
            compute(t1, xb1, yb1)
            return carry
        lax.fori_loop(0, nch // 2, pair, 0)
        pltpu.sync_copy(pb, out_hbm.at[pl.ds(w * nch, nch)])

    return k4(i0r, i1or, X, ab).reshape(B)


def kernel(idxs, feats, ifeats, keys, ikeys, values, scale):
    i0 = idxs[0].astype(jnp.int32)
    i1 = idxs[1].astype(jnp.int32)
    i0r = i0.reshape(B // CH, CH)
    i1r = i1.reshape(B // CH, CH)
    i1or = (i1 + D0).reshape(B // CH, CH)

    ikeys_t = jnp.stack([ikeys[0].T, ikeys[1].T])
    keys_t = jnp.stack([keys[0].T, keys[1].T])

    qs = _quantize(ifeats, ikeys_t, aug=True)
    zeros = jnp.zeros((CH, FA), jnp.float32)
    acc = _k2_call(i0r, i1r, i1or, qs, zeros)
    p = _quantize(feats, keys_t, aug=False)

    w = values[0]
    sc = scale[0]
    cAB = w[0] - w[1] - w[2] + w[3]
    cA = w[1] - w[3]
    cB = w[2] - w[3]
    C = jnp.sum(w[3])

    c1 = jnp.stack([(sc * cAB)[None, :], jnp.ones((1, F), jnp.float32)])
    c2 = jnp.stack([(sc * cA)[None, :], (sc * cB)[None, :]])
    c3 = jnp.stack([(sc * C)[None, None], jnp.zeros((1, 1), jnp.float32)])

    X, ab = _k3_call(p, acc, c1, c2, c3)
    preds = _k4_call(i0r, i1or, X, ab[:, 0])
    return preds
